# speedup vs baseline: 7.4074x; 7.4074x over previous
import jax
import jax.numpy as jnp
from jax.experimental import pallas as pl
from jax.experimental.pallas import tpu as pltpu
from jax.experimental.pallas import tpu_sc as plsc

BATCH = 4096
HIST = 200
EMBED_DIM = 128
NUM_IDX = BATCH * HIST

WINDOW = 128

_MESH = plsc.VectorSubcoreMesh(core_axis_name="c", subcore_axis_name="s")


def _gather_kernel(table_hbm, idx_hbm, out_hbm):
    def body(idx_vmem, out_vmem):
        pltpu.sync_copy(table_hbm.at[idx_vmem.at[0]], out_vmem)

    pltpu.emit_pipeline(
        body,
        grid=(NUM_IDX // WINDOW,),
        in_specs=[pl.BlockSpec((1, WINDOW), index_map=lambda i: (0, i))],
        out_specs=[pl.BlockSpec((WINDOW, EMBED_DIM), index_map=lambda i: (i, 0))],
        core_axis_name=("c", "s"),
        dimension_semantics=(pltpu.PARALLEL,),
    )(idx_hbm, out_hbm)


def kernel(x, table):
    idx = x.reshape(1, NUM_IDX).astype(jnp.int32)
    run = pl.kernel(
        _gather_kernel,
        out_type=jax.ShapeDtypeStruct((NUM_IDX, EMBED_DIM), table.dtype),
        mesh=_MESH,
    )
    out = run(table, idx)
    return out.reshape(BATCH, HIST, EMBED_DIM)


# speedup vs baseline: 7.8663x; 1.0620x over previous
import jax
import jax.numpy as jnp
from jax.experimental import pallas as pl
from jax.experimental.pallas import tpu as pltpu
from jax.experimental.pallas import tpu_sc as plsc

BATCH = 4096
HIST = 200
EMBED_DIM = 128
NUM_IDX = BATCH * HIST

WINDOW = 256
SUB = 128

_MESH = plsc.VectorSubcoreMesh(core_axis_name="c", subcore_axis_name="s")


def _gather_kernel(table_hbm, idx_hbm, out_hbm):
    def body(idx_vmem, out_vmem):
        for j in range(WINDOW // SUB):
            pltpu.sync_copy(
                table_hbm.at[idx_vmem.at[0].at[pl.ds(j * SUB, SUB)]],
                out_vmem.at[pl.ds(j * SUB, SUB)],
            )

    pltpu.emit_pipeline(
        body,
        grid=(NUM_IDX // WINDOW,),
        in_specs=[pl.BlockSpec((1, WINDOW), index_map=lambda i: (0, i))],
        out_specs=[pl.BlockSpec((WINDOW, EMBED_DIM), index_map=lambda i: (i, 0))],
        core_axis_name=("c", "s"),
        dimension_semantics=(pltpu.PARALLEL,),
    )(idx_hbm, out_hbm)


def kernel(x, table):
    idx = x.reshape(1, NUM_IDX).astype(jnp.int32)
    run = pl.kernel(
        _gather_kernel,
        out_type=jax.ShapeDtypeStruct((NUM_IDX, EMBED_DIM), table.dtype),
        mesh=_MESH,
    )
    out = run(table, idx)
    return out.reshape(BATCH, HIST, EMBED_DIM)
